# Initial kernel scaffold; baseline (speedup 1.0000x reference)
import jax
import jax.numpy as jnp
from jax.experimental import pallas as pl


def kernel(category, sub_category, element, brand, product_id_remapped, price_tensor, edge_index, batch, emb_cat, emb_sub, emb_elem, emb_brand, emb_item, W_msg, b_msg, W_ih, W_hh, b_ih, b_hh, gate_W1, gate_b1, gate_W2, gate_b2, W_fc, b_fc):
    raise NotImplementedError("write your pallas kernel here")


# speedup vs baseline: 3.7517x; 3.7517x over previous
import functools

import jax
import jax.numpy as jnp
from jax import lax
from jax.experimental import pallas as pl
from jax.experimental.pallas import tpu as pltpu
from jax.experimental.pallas import tpu_sc as plsc

N = 50000
E = 800000
B = 512
HID = 100
EMB = 16
N_ITEMS = 100000

NPAD = 53248
NBLK = 1024
NGRID = NPAD // NBLK
IDX_ROWS = NPAD // 128
ROWS_PER_W = IDX_ROWS // 32

G = 7
ACC_ROWS = NPAD + 256
STRIPE = ACC_ROWS // 16

EPAD = 802816
E_ROWS = EPAD // 128
E_ROWS_CORE = E_ROWS // 2
E_ROWS_TILE = E_ROWS_CORE // 16
KB = 7
OB = E_ROWS_TILE // KB

VBLK = 2048
VGRID = 49
VPAD = VGRID * VBLK

_mesh = plsc.VectorSubcoreMesh(core_axis_name="c", subcore_axis_name="s")
_sc_params = pltpu.CompilerParams(use_tc_tiling_on_sc=False)


@functools.partial(
    pl.kernel,
    out_type=jax.ShapeDtypeStruct((NPAD, 128), jnp.float32),
    mesh=_mesh,
    compiler_params=_sc_params,
    scratch_types=[
        pltpu.VMEM((ROWS_PER_W, 128), jnp.int32),
        pltpu.VMEM((ROWS_PER_W * 128, 16), jnp.float32),
        pltpu.SemaphoreType.DMA,
    ],
)
def _emb_gather(t0, t1, t2, t3, t4, idx5, out, idxv, rowsv, sem):
    c = lax.axis_index("c")
    s = lax.axis_index("s")
    wid = s * 2 + c
    rbase = wid * ROWS_PER_W
    for t, tab in enumerate([t0, t1, t2, t3, t4]):
        pltpu.sync_copy(idx5.at[t, pl.ds(rbase, ROWS_PER_W)], idxv)
        descs = []
        for j in range(ROWS_PER_W):
            descs.append(
                pltpu.async_copy(tab.at[idxv.at[j]],
                                 rowsv.at[pl.ds(j * 128, 128)], sem))
        for d in descs:
            d.wait()
        pltpu.sync_copy(
            rowsv,
            out.at[pl.ds(rbase * 128, ROWS_PER_W * 128), pl.ds(t * 16, 16)])


@functools.partial(
    pl.kernel,
    out_type=jax.ShapeDtypeStruct((2, ACC_ROWS, 128), jnp.float32),
    mesh=_mesh,
    compiler_params=_sc_params,
    scratch_types=[
        pltpu.VMEM((KB, 128), jnp.int32),
        pltpu.VMEM((KB, 128), jnp.int32),
        pltpu.VMEM((128, 16), jnp.float32),
        pltpu.VMEM((128, 16), jnp.float32),
        pltpu.VMEM_SHARED((ACC_ROWS, 16), jnp.float32),
        pltpu.SemaphoreType.DMA,
    ],
)
def _edge_agg(h7, srcm, dstm, zeros, out, srcv, dstv, r0, r1, acc, sem):
    c = lax.axis_index("c")
    s = lax.axis_index("s")
    rbase = c * E_ROWS_CORE + s * E_ROWS_TILE
    rows = [r0, r1]
    for g in range(G):
        pltpu.sync_copy(zeros, acc.at[pl.ds(s * STRIPE, STRIPE)])
        plsc.subcore_barrier()

        def ob_body(ob, _):
            pltpu.sync_copy(srcm.at[pl.ds(rbase + ob * KB, KB)], srcv)
            pltpu.sync_copy(dstm.at[pl.ds(rbase + ob * KB, KB)], dstv)
            d = pltpu.async_copy(h7.at[g].at[srcv.at[0]], rows[0], sem)
            for j in range(1, KB):
                d_next = pltpu.async_copy(h7.at[g].at[srcv.at[j]],
                                          rows[j % 2], sem)
                d.wait()
                pltpu.sync_copy(rows[(j - 1) % 2], acc.at[dstv.at[j - 1]],
                                add=True)
                d = d_next
            d.wait()
            pltpu.sync_copy(rows[(KB - 1) % 2], acc.at[dstv.at[KB - 1]],
                            add=True)
            return _

        lax.fori_loop(0, OB, ob_body, None)
        plsc.subcore_barrier()
        pltpu.sync_copy(acc.at[pl.ds(s * STRIPE, STRIPE)],
                        out.at[c, pl.ds(s * STRIPE, STRIPE),
                               pl.ds(g * 16, 16)])
        plsc.subcore_barrier()


def _hmsg_body(xe_ref, price_ref, wp_ref, wall_ref, b_ref, out_ref):
    lanes = lax.broadcasted_iota(jnp.int32, (NBLK, 128), 1)
    xz = jnp.where(lanes < 80, xe_ref[...], 0.0)
    out_ref[...] = (price_ref[...] * wp_ref[...] + b_ref[...]
                    + jnp.dot(xz, wall_ref[...],
                              preferred_element_type=jnp.float32))


def _hmsg(xe, price, wp, wall, b):
    return pl.pallas_call(
        _hmsg_body,
        grid=(NGRID,),
        in_specs=[
            pl.BlockSpec((NBLK, 128), lambda i: (i, 0)),
            pl.BlockSpec((NBLK, 1), lambda i: (i, 0)),
            pl.BlockSpec((1, 128), lambda i: (0, 0)),
            pl.BlockSpec((128, 128), lambda i: (0, 0)),
            pl.BlockSpec((1, 128), lambda i: (0, 0)),
        ],
        out_specs=pl.BlockSpec((NBLK, 128), lambda i: (i, 0)),
        out_shape=jax.ShapeDtypeStruct((NPAD, 128), jnp.float32),
    )(xe, price, wp, wall, b)


def _gru_body(ms_ref, h_ref, batch_ref, wih_ref, whh_ref, bih_ref, bhh_ref,
              w1_ref, b1_ref, w2_ref, hq_ref, gl_ref, stats_ref,
              gmax_scr, den_scr):
    i = pl.program_id(0)
    lanes = lax.broadcasted_iota(jnp.int32, (NBLK, 128), 1)
    m = ms_ref[0] + ms_ref[1]
    deg = m[:, 100:101]
    m128 = jnp.where(lanes < HID, m, 0.0) / jnp.maximum(deg, 1.0)
    h0 = jnp.where(lanes < HID, h_ref[...], 0.0)

    gi = jnp.dot(m128, wih_ref[0], preferred_element_type=jnp.float32)
    gh = jnp.dot(h0, whh_ref[0], preferred_element_type=jnp.float32)
    gi2 = jnp.dot(m128, wih_ref[1], preferred_element_type=jnp.float32)
    gh2 = jnp.dot(h0, whh_ref[1], preferred_element_type=jnp.float32)
    gi3 = jnp.dot(m128, wih_ref[2], preferred_element_type=jnp.float32)
    gh3 = jnp.dot(h0, whh_ref[2], preferred_element_type=jnp.float32)
    r = jax.nn.sigmoid(gi + bih_ref[0] + gh + bhh_ref[0])
    z = jax.nn.sigmoid(gi2 + bih_ref[1] + gh2 + bhh_ref[1])
    n = jnp.tanh(gi3 + bih_ref[2] + r * (gh3 + bhh_ref[2]))
    hn = (1.0 - z) * n + z * h0
    hq_ref[...] = hn

    q = jnp.maximum(jnp.dot(hn, w1_ref[...],
                            preferred_element_type=jnp.float32)
                    + b1_ref[...], 0.0)
    gl = jnp.sum(q * w2_ref[...], axis=1, keepdims=True)
    gl_ref[...] = gl

    seg = lax.broadcasted_iota(jnp.int32, (NBLK, B), 1)
    mask = batch_ref[...] == seg
    glb = jnp.broadcast_to(gl, (NBLK, B))
    neg = jnp.float32(-jnp.inf)

    @pl.when(i == 0)
    def _():
        gmax_scr[...] = jnp.full((1, B), neg, jnp.float32)
        den_scr[...] = jnp.zeros((1, B), jnp.float32)

    bmax = jnp.max(jnp.where(mask, glb, neg), axis=0, keepdims=True)
    old = gmax_scr[...]
    new = jnp.maximum(old, bmax)
    scale = jnp.where(old == neg, 0.0, jnp.exp(old - new))
    ex = jnp.where(mask, jnp.exp(glb - jnp.broadcast_to(new, (NBLK, B))), 0.0)
    den_scr[...] = den_scr[...] * scale + jnp.sum(ex, axis=0, keepdims=True)
    gmax_scr[...] = new

    @pl.when(i == NGRID - 1)
    def _():
        gm = gmax_scr[...]
        stats_ref[0:1, :] = jnp.where(gm == neg, 0.0, gm)
        stats_ref[1:2, :] = den_scr[...]


def _gru_gate(ms, h, batch, wih, whh, bih, bhh, w1, b1, w2):
    return pl.pallas_call(
        _gru_body,
        grid=(NGRID,),
        in_specs=[
            pl.BlockSpec((2, NBLK, 128), lambda i: (0, i, 0)),
            pl.BlockSpec((NBLK, 128), lambda i: (i, 0)),
            pl.BlockSpec((NBLK, 1), lambda i: (i, 0)),
            pl.BlockSpec((3, 128, 128), lambda i: (0, 0, 0)),
            pl.BlockSpec((3, 128, 128), lambda i: (0, 0, 0)),
            pl.BlockSpec((3, 1, 128), lambda i: (0, 0, 0)),
            pl.BlockSpec((3, 1, 128), lambda i: (0, 0, 0)),
            pl.BlockSpec((128, 128), lambda i: (0, 0)),
            pl.BlockSpec((1, 128), lambda i: (0, 0)),
            pl.BlockSpec((1, 128), lambda i: (0, 0)),
        ],
        out_specs=[
            pl.BlockSpec((NBLK, 128), lambda i: (i, 0)),
            pl.BlockSpec((NBLK, 1), lambda i: (i, 0)),
            pl.BlockSpec((2, B), lambda i: (0, 0)),
        ],
        out_shape=[
            jax.ShapeDtypeStruct((NPAD, 128), jnp.float32),
            jax.ShapeDtypeStruct((NPAD, 1), jnp.float32),
            jax.ShapeDtypeStruct((2, B), jnp.float32),
        ],
        scratch_shapes=[
            pltpu.VMEM((1, B), jnp.float32),
            pltpu.VMEM((1, B), jnp.float32),
        ],
    )(ms, h, batch, wih, whh, bih, bhh, w1, b1, w2)


def _pool_body(hq_ref, gl_ref, batch_ref, stats_ref, out_ref, acc_scr):
    i = pl.program_id(0)
    seg = lax.broadcasted_iota(jnp.int32, (NBLK, B), 1)
    mask = batch_ref[...] == seg
    neg = jnp.float32(-jnp.inf)
    gmax_r = jnp.broadcast_to(stats_ref[0:1, :], (NBLK, B))
    den_r = jnp.broadcast_to(stats_ref[1:2, :], (NBLK, B))
    gmax_i = jnp.max(jnp.where(mask, gmax_r, neg), axis=1, keepdims=True)
    den_i = jnp.max(jnp.where(mask, den_r, neg), axis=1, keepdims=True)
    alpha = jnp.exp(gl_ref[...] - gmax_i) / jnp.maximum(den_i, 1e-16)
    w = jnp.where(mask, jnp.broadcast_to(alpha, (NBLK, B)), 0.0)

    @pl.when(i == 0)
    def _():
        acc_scr[...] = jnp.zeros((B, 128), jnp.float32)

    acc_scr[...] += lax.dot_general(w, hq_ref[...], (((0,), (0,)), ((), ())),
                                    preferred_element_type=jnp.float32)

    @pl.when(i == NGRID - 1)
    def _():
        out_ref[...] = acc_scr[...]


def _pool(hq, gl, batch, stats):
    return pl.pallas_call(
        _pool_body,
        grid=(NGRID,),
        in_specs=[
            pl.BlockSpec((NBLK, 128), lambda i: (i, 0)),
            pl.BlockSpec((NBLK, 1), lambda i: (i, 0)),
            pl.BlockSpec((NBLK, 1), lambda i: (i, 0)),
            pl.BlockSpec((2, B), lambda i: (0, 0)),
        ],
        out_specs=pl.BlockSpec((B, 128), lambda i: (0, 0)),
        out_shape=jax.ShapeDtypeStruct((B, 128), jnp.float32),
        scratch_shapes=[pltpu.VMEM((B, 128), jnp.float32)],
    )(hq, gl, batch, stats)


def _fc_body(p_ref, w_ref, b_ref, out_ref):
    out_ref[...] = (jnp.dot(p_ref[...], w_ref[...],
                            preferred_element_type=jnp.float32)
                    + b_ref[...])


def _fc(pooled, wt, b):
    return pl.pallas_call(
        _fc_body,
        grid=(VGRID,),
        in_specs=[
            pl.BlockSpec((B, 128), lambda i: (0, 0)),
            pl.BlockSpec((128, VBLK), lambda i: (0, i)),
            pl.BlockSpec((1, VBLK), lambda i: (0, i)),
        ],
        out_specs=pl.BlockSpec((B, VBLK), lambda i: (0, i)),
        out_shape=jax.ShapeDtypeStruct((B, N_ITEMS), jnp.float32),
    )(pooled, wt, b)


def kernel(category, sub_category, element, brand, product_id_remapped,
           price_tensor, edge_index, batch,
           emb_cat, emb_sub, emb_elem, emb_brand, emb_item,
           W_msg, b_msg, W_ih, W_hh, b_ih, b_hh,
           gate_W1, gate_b1, gate_W2, gate_b2, W_fc, b_fc):
    f32 = jnp.float32
    npd = NPAD - N
    pad_fill = jnp.arange(npd, dtype=jnp.int32)
    idxs = []
    for arr, size in ((category, 1000), (sub_category, 5000),
                      (element, 10000), (brand, 20000),
                      (product_id_remapped, N_ITEMS)):
        idxs.append(jnp.concatenate([arr.astype(jnp.int32), pad_fill % size]))
    idx5 = jnp.stack(idxs).reshape(5, IDX_ROWS, 128)

    xe = _emb_gather(emb_cat, emb_sub, emb_elem, emb_brand, emb_item, idx5)

    price = jnp.concatenate([price_tensor.astype(f32),
                             jnp.zeros((npd, 1), f32)])
    wp = jnp.zeros((1, 128), f32).at[0, :HID].set(W_msg[:, 0])
    wall = jnp.zeros((128, 128), f32).at[:80, :HID].set(W_msg[:, 1:].T)
    bp = jnp.zeros((1, 128), f32).at[0, :HID].set(b_msg).at[0, HID].set(1.0)
    h_pad = _hmsg(xe, price, wp, wall, bp)

    h7 = jnp.transpose(h_pad[:, :G * 16].reshape(NPAD, G, 16), (1, 0, 2))

    epd = EPAD - E
    src = jnp.concatenate([edge_index[0].astype(jnp.int32),
                           jnp.arange(epd, dtype=jnp.int32) % N])
    dst = jnp.concatenate([edge_index[1].astype(jnp.int32),
                           NPAD + (jnp.arange(epd, dtype=jnp.int32) % 256)])
    srcm = src.reshape(E_ROWS, 128)
    dstm = dst.reshape(E_ROWS, 128)
    zeros = jnp.zeros((STRIPE, 16), f32)
    ms = _edge_agg(h7, srcm, dstm, zeros)

    batch_pad = jnp.concatenate([batch.astype(jnp.int32),
                                 jnp.full((npd,), B, jnp.int32)])
    batch2 = batch_pad.reshape(NPAD, 1)

    def pad3(w, bvec):
        wm = jnp.zeros((3, 128, 128), f32)
        wm = wm.at[:, :HID, :HID].set(
            jnp.transpose(w.reshape(3, HID, HID), (0, 2, 1)))
        bm = jnp.zeros((3, 1, 128), f32).at[:, 0, :HID].set(
            bvec.reshape(3, HID))
        return wm, bm

    wih, bih = pad3(W_ih, b_ih)
    whh, bhh = pad3(W_hh, b_hh)
    w1 = jnp.zeros((128, 128), f32).at[:HID, :HID].set(gate_W1.T)
    b1 = jnp.zeros((1, 128), f32).at[0, :HID].set(gate_b1)
    w2 = jnp.zeros((1, 128), f32).at[0, :HID].set(gate_W2[0])

    hq, gl, stats = _gru_gate(ms, h_pad, batch2, wih, whh, bih, bhh,
                              w1, b1, w2)
    pooled = _pool(hq, gl, batch2, stats)

    wfct = jnp.zeros((128, VPAD), f32).at[:HID, :N_ITEMS].set(W_fc.T)
    bfc = jnp.zeros((1, VPAD), f32).at[0, :N_ITEMS].set(b_fc)
    return _fc(pooled, wfct, bfc)


# speedup vs baseline: 5.5763x; 1.4864x over previous
import functools

import jax
import jax.numpy as jnp
from jax import lax
from jax.experimental import pallas as pl
from jax.experimental.pallas import tpu as pltpu
from jax.experimental.pallas import tpu_sc as plsc

N = 50000
E = 800000
B = 512
HID = 100
EMB = 16
N_ITEMS = 100000

NPAD = 53248
NBLK = 1024
NGRID = NPAD // NBLK
IDX_ROWS = NPAD // 128
ROWS_PER_W = IDX_ROWS // 32

G = 7
ACC_ROWS = NPAD + 256
STRIPE = ACC_ROWS // 16

EPAD = 802816
E_ROWS = EPAD // 128
E_ROWS_CORE = E_ROWS // 2
E_ROWS_TILE = E_ROWS_CORE // 16
KB = 7
OB = E_ROWS_TILE // KB

VBLK = 2048
VGRID = 49
VPAD = VGRID * VBLK

_mesh = plsc.VectorSubcoreMesh(core_axis_name="c", subcore_axis_name="s")
_sc_params = pltpu.CompilerParams(use_tc_tiling_on_sc=False)


@functools.partial(
    pl.kernel,
    out_type=jax.ShapeDtypeStruct((NPAD, 128), jnp.float32),
    mesh=_mesh,
    compiler_params=_sc_params,
    scratch_types=[
        pltpu.VMEM((ROWS_PER_W, 128), jnp.int32),
        pltpu.VMEM((ROWS_PER_W * 128, 16), jnp.float32),
        pltpu.SemaphoreType.DMA,
    ],
)
def _emb_gather(t0, t1, t2, t3, t4, idx5, out, idxv, rowsv, sem):
    c = lax.axis_index("c")
    s = lax.axis_index("s")
    wid = s * 2 + c
    rbase = wid * ROWS_PER_W
    for t, tab in enumerate([t0, t1, t2, t3, t4]):
        pltpu.sync_copy(idx5.at[t, pl.ds(rbase, ROWS_PER_W)], idxv)
        descs = []
        for j in range(ROWS_PER_W):
            descs.append(
                pltpu.async_copy(tab.at[idxv.at[j]],
                                 rowsv.at[pl.ds(j * 128, 128)], sem))
        for d in descs:
            d.wait()
        pltpu.sync_copy(
            rowsv,
            out.at[pl.ds(rbase * 128, ROWS_PER_W * 128), pl.ds(t * 16, 16)])


@functools.partial(
    pl.kernel,
    out_type=jax.ShapeDtypeStruct((2, ACC_ROWS, 128), jnp.float32),
    mesh=_mesh,
    compiler_params=_sc_params,
    scratch_types=[
        pltpu.VMEM((E_ROWS_TILE, 128), jnp.int32),
        pltpu.VMEM((E_ROWS_TILE, 128), jnp.int32),
    ] + [pltpu.VMEM((128, 16), jnp.float32) for _ in range(KB)] + [
        pltpu.VMEM_SHARED((ACC_ROWS, 16), jnp.float32),
        pltpu.SemaphoreType.DMA((KB,)),
        pltpu.SemaphoreType.DMA((KB,)),
    ],
)
def _edge_agg(h7, srcm, dstm, zeros, out, srcv, dstv,
              r0, r1, r2, r3, r4, r5, r6, acc, semg, sems):
    c = lax.axis_index("c")
    s = lax.axis_index("s")
    rbase = c * E_ROWS_CORE + s * E_ROWS_TILE
    rows = [r0, r1, r2, r3, r4, r5, r6]
    pltpu.sync_copy(srcm.at[pl.ds(rbase, E_ROWS_TILE)], srcv)
    pltpu.sync_copy(dstm.at[pl.ds(rbase, E_ROWS_TILE)], dstv)
    for g in range(G):
        pltpu.sync_copy(zeros, acc.at[pl.ds(s * STRIPE, STRIPE)])
        plsc.subcore_barrier()

        def ob_body(ob, _):
            @pl.when(ob > 0)
            def _():
                for j in range(KB):
                    pltpu.make_async_copy(
                        rows[j], acc.at[dstv.at[0]], sems.at[j]).wait()
            gd = []
            for j in range(KB):
                gd.append(pltpu.async_copy(
                    h7.at[g].at[srcv.at[ob * KB + j]], rows[j], semg.at[j]))
            for j in range(KB):
                gd[j].wait()
                pltpu.async_copy(rows[j], acc.at[dstv.at[ob * KB + j]],
                                 sems.at[j], add=True)
            return _

        lax.fori_loop(0, OB, ob_body, None)
        for j in range(KB):
            pltpu.make_async_copy(rows[j], acc.at[dstv.at[0]],
                                  sems.at[j]).wait()
        plsc.subcore_barrier()
        pltpu.sync_copy(acc.at[pl.ds(s * STRIPE, STRIPE)],
                        out.at[c, pl.ds(s * STRIPE, STRIPE),
                               pl.ds(g * 16, 16)])
        plsc.subcore_barrier()


def _hmsg_body(xe_ref, price_ref, wp_ref, wall_ref, b_ref, out_ref):
    lanes = lax.broadcasted_iota(jnp.int32, (NBLK, 128), 1)
    xz = jnp.where(lanes < 80, xe_ref[...], 0.0)
    out_ref[...] = (price_ref[...] * wp_ref[...] + b_ref[...]
                    + jnp.dot(xz, wall_ref[...],
                              preferred_element_type=jnp.float32))


def _hmsg(xe, price, wp, wall, b):
    return pl.pallas_call(
        _hmsg_body,
        grid=(NGRID,),
        in_specs=[
            pl.BlockSpec((NBLK, 128), lambda i: (i, 0)),
            pl.BlockSpec((NBLK, 1), lambda i: (i, 0)),
            pl.BlockSpec((1, 128), lambda i: (0, 0)),
            pl.BlockSpec((128, 128), lambda i: (0, 0)),
            pl.BlockSpec((1, 128), lambda i: (0, 0)),
        ],
        out_specs=pl.BlockSpec((NBLK, 128), lambda i: (i, 0)),
        out_shape=jax.ShapeDtypeStruct((NPAD, 128), jnp.float32),
    )(xe, price, wp, wall, b)


def _gru_body(ms_ref, h_ref, batch_ref, wih_ref, whh_ref, bih_ref, bhh_ref,
              w1_ref, b1_ref, w2_ref, hq_ref, gl_ref, stats_ref,
              gmax_scr, den_scr):
    i = pl.program_id(0)
    lanes = lax.broadcasted_iota(jnp.int32, (NBLK, 128), 1)
    m = ms_ref[0] + ms_ref[1]
    deg = m[:, 100:101]
    m128 = jnp.where(lanes < HID, m, 0.0) / jnp.maximum(deg, 1.0)
    h0 = jnp.where(lanes < HID, h_ref[...], 0.0)

    gi = jnp.dot(m128, wih_ref[0], preferred_element_type=jnp.float32)
    gh = jnp.dot(h0, whh_ref[0], preferred_element_type=jnp.float32)
    gi2 = jnp.dot(m128, wih_ref[1], preferred_element_type=jnp.float32)
    gh2 = jnp.dot(h0, whh_ref[1], preferred_element_type=jnp.float32)
    gi3 = jnp.dot(m128, wih_ref[2], preferred_element_type=jnp.float32)
    gh3 = jnp.dot(h0, whh_ref[2], preferred_element_type=jnp.float32)
    r = jax.nn.sigmoid(gi + bih_ref[0] + gh + bhh_ref[0])
    z = jax.nn.sigmoid(gi2 + bih_ref[1] + gh2 + bhh_ref[1])
    n = jnp.tanh(gi3 + bih_ref[2] + r * (gh3 + bhh_ref[2]))
    hn = (1.0 - z) * n + z * h0
    hq_ref[...] = hn

    q = jnp.maximum(jnp.dot(hn, w1_ref[...],
                            preferred_element_type=jnp.float32)
                    + b1_ref[...], 0.0)
    gl = jnp.sum(q * w2_ref[...], axis=1, keepdims=True)
    gl_ref[...] = gl

    seg = lax.broadcasted_iota(jnp.int32, (NBLK, B), 1)
    mask = batch_ref[...] == seg
    glb = jnp.broadcast_to(gl, (NBLK, B))
    neg = jnp.float32(-jnp.inf)

    @pl.when(i == 0)
    def _():
        gmax_scr[...] = jnp.full((1, B), neg, jnp.float32)
        den_scr[...] = jnp.zeros((1, B), jnp.float32)

    bmax = jnp.max(jnp.where(mask, glb, neg), axis=0, keepdims=True)
    old = gmax_scr[...]
    new = jnp.maximum(old, bmax)
    scale = jnp.where(old == neg, 0.0, jnp.exp(old - new))
    ex = jnp.where(mask, jnp.exp(glb - jnp.broadcast_to(new, (NBLK, B))), 0.0)
    den_scr[...] = den_scr[...] * scale + jnp.sum(ex, axis=0, keepdims=True)
    gmax_scr[...] = new

    @pl.when(i == NGRID - 1)
    def _():
        gm = gmax_scr[...]
        stats_ref[0:1, :] = jnp.where(gm == neg, 0.0, gm)
        stats_ref[1:2, :] = den_scr[...]


def _gru_gate(ms, h, batch, wih, whh, bih, bhh, w1, b1, w2):
    return pl.pallas_call(
        _gru_body,
        grid=(NGRID,),
        in_specs=[
            pl.BlockSpec((2, NBLK, 128), lambda i: (0, i, 0)),
            pl.BlockSpec((NBLK, 128), lambda i: (i, 0)),
            pl.BlockSpec((NBLK, 1), lambda i: (i, 0)),
            pl.BlockSpec((3, 128, 128), lambda i: (0, 0, 0)),
            pl.BlockSpec((3, 128, 128), lambda i: (0, 0, 0)),
            pl.BlockSpec((3, 1, 128), lambda i: (0, 0, 0)),
            pl.BlockSpec((3, 1, 128), lambda i: (0, 0, 0)),
            pl.BlockSpec((128, 128), lambda i: (0, 0)),
            pl.BlockSpec((1, 128), lambda i: (0, 0)),
            pl.BlockSpec((1, 128), lambda i: (0, 0)),
        ],
        out_specs=[
            pl.BlockSpec((NBLK, 128), lambda i: (i, 0)),
            pl.BlockSpec((NBLK, 1), lambda i: (i, 0)),
            pl.BlockSpec((2, B), lambda i: (0, 0)),
        ],
        out_shape=[
            jax.ShapeDtypeStruct((NPAD, 128), jnp.float32),
            jax.ShapeDtypeStruct((NPAD, 1), jnp.float32),
            jax.ShapeDtypeStruct((2, B), jnp.float32),
        ],
        scratch_shapes=[
            pltpu.VMEM((1, B), jnp.float32),
            pltpu.VMEM((1, B), jnp.float32),
        ],
    )(ms, h, batch, wih, whh, bih, bhh, w1, b1, w2)


def _pool_body(hq_ref, gl_ref, batch_ref, stats_ref, out_ref, acc_scr):
    i = pl.program_id(0)
    seg = lax.broadcasted_iota(jnp.int32, (NBLK, B), 1)
    mask = batch_ref[...] == seg
    neg = jnp.float32(-jnp.inf)
    gmax_r = jnp.broadcast_to(stats_ref[0:1, :], (NBLK, B))
    den_r = jnp.broadcast_to(stats_ref[1:2, :], (NBLK, B))
    gmax_i = jnp.max(jnp.where(mask, gmax_r, neg), axis=1, keepdims=True)
    den_i = jnp.max(jnp.where(mask, den_r, neg), axis=1, keepdims=True)
    alpha = jnp.exp(gl_ref[...] - gmax_i) / jnp.maximum(den_i, 1e-16)
    w = jnp.where(mask, jnp.broadcast_to(alpha, (NBLK, B)), 0.0)

    @pl.when(i == 0)
    def _():
        acc_scr[...] = jnp.zeros((B, 128), jnp.float32)

    acc_scr[...] += lax.dot_general(w, hq_ref[...], (((0,), (0,)), ((), ())),
                                    preferred_element_type=jnp.float32)

    @pl.when(i == NGRID - 1)
    def _():
        out_ref[...] = acc_scr[...]


def _pool(hq, gl, batch, stats):
    return pl.pallas_call(
        _pool_body,
        grid=(NGRID,),
        in_specs=[
            pl.BlockSpec((NBLK, 128), lambda i: (i, 0)),
            pl.BlockSpec((NBLK, 1), lambda i: (i, 0)),
            pl.BlockSpec((NBLK, 1), lambda i: (i, 0)),
            pl.BlockSpec((2, B), lambda i: (0, 0)),
        ],
        out_specs=pl.BlockSpec((B, 128), lambda i: (0, 0)),
        out_shape=jax.ShapeDtypeStruct((B, 128), jnp.float32),
        scratch_shapes=[pltpu.VMEM((B, 128), jnp.float32)],
    )(hq, gl, batch, stats)


def _fc_body(p_ref, w_ref, b_ref, out_ref):
    out_ref[...] = (lax.dot_general(p_ref[...][:, :HID], w_ref[...],
                                    (((1,), (1,)), ((), ())),
                                    preferred_element_type=jnp.float32)
                    + b_ref[...])


def _fc(pooled, w, b):
    return pl.pallas_call(
        _fc_body,
        grid=(VGRID,),
        in_specs=[
            pl.BlockSpec((B, 128), lambda i: (0, 0)),
            pl.BlockSpec((VBLK, HID), lambda i: (i, 0)),
            pl.BlockSpec((1, VBLK), lambda i: (0, i)),
        ],
        out_specs=pl.BlockSpec((B, VBLK), lambda i: (0, i)),
        out_shape=jax.ShapeDtypeStruct((B, N_ITEMS), jnp.float32),
    )(pooled, w, b)


def kernel(category, sub_category, element, brand, product_id_remapped,
           price_tensor, edge_index, batch,
           emb_cat, emb_sub, emb_elem, emb_brand, emb_item,
           W_msg, b_msg, W_ih, W_hh, b_ih, b_hh,
           gate_W1, gate_b1, gate_W2, gate_b2, W_fc, b_fc):
    f32 = jnp.float32
    npd = NPAD - N
    pad_fill = jnp.arange(npd, dtype=jnp.int32)
    idxs = []
    for arr, size in ((category, 1000), (sub_category, 5000),
                      (element, 10000), (brand, 20000),
                      (product_id_remapped, N_ITEMS)):
        idxs.append(jnp.concatenate([arr.astype(jnp.int32), pad_fill % size]))
    idx5 = jnp.stack(idxs).reshape(5, IDX_ROWS, 128)

    xe = _emb_gather(emb_cat, emb_sub, emb_elem, emb_brand, emb_item, idx5)

    price = jnp.concatenate([price_tensor.astype(f32),
                             jnp.zeros((npd, 1), f32)])
    wp = jnp.zeros((1, 128), f32).at[0, :HID].set(W_msg[:, 0])
    wall = jnp.zeros((128, 128), f32).at[:80, :HID].set(W_msg[:, 1:].T)
    bp = jnp.zeros((1, 128), f32).at[0, :HID].set(b_msg).at[0, HID].set(1.0)
    h_pad = _hmsg(xe, price, wp, wall, bp)

    h7 = jnp.transpose(h_pad[:, :G * 16].reshape(NPAD, G, 16), (1, 0, 2))

    epd = EPAD - E
    src = jnp.concatenate([edge_index[0].astype(jnp.int32),
                           jnp.arange(epd, dtype=jnp.int32) % N])
    dst = jnp.concatenate([edge_index[1].astype(jnp.int32),
                           NPAD + (jnp.arange(epd, dtype=jnp.int32) % 256)])
    srcm = src.reshape(E_ROWS, 128)
    dstm = dst.reshape(E_ROWS, 128)
    zeros = jnp.zeros((STRIPE, 16), f32)
    ms = _edge_agg(h7, srcm, dstm, zeros)

    batch_pad = jnp.concatenate([batch.astype(jnp.int32),
                                 jnp.full((npd,), B, jnp.int32)])
    batch2 = batch_pad.reshape(NPAD, 1)

    def pad3(w, bvec):
        wm = jnp.zeros((3, 128, 128), f32)
        wm = wm.at[:, :HID, :HID].set(
            jnp.transpose(w.reshape(3, HID, HID), (0, 2, 1)))
        bm = jnp.zeros((3, 1, 128), f32).at[:, 0, :HID].set(
            bvec.reshape(3, HID))
        return wm, bm

    wih, bih = pad3(W_ih, b_ih)
    whh, bhh = pad3(W_hh, b_hh)
    w1 = jnp.zeros((128, 128), f32).at[:HID, :HID].set(gate_W1.T)
    b1 = jnp.zeros((1, 128), f32).at[0, :HID].set(gate_b1)
    w2 = jnp.zeros((1, 128), f32).at[0, :HID].set(gate_W2[0])

    hq, gl, stats = _gru_gate(ms, h_pad, batch2, wih, whh, bih, bhh,
                              w1, b1, w2)
    pooled = _pool(hq, gl, batch2, stats)
    return _fc(pooled, W_fc, b_fc.reshape(1, N_ITEMS))
